# baseline (speedup 1.0000x reference)
import functools

import jax
import jax.numpy as jnp
from jax import lax
from jax.experimental import pallas as pl
from jax.experimental.pallas import tpu as pltpu
from jax.experimental.pallas import tpu_sc as plsc

_BATCH = 16384
_FEATS = 64


def _sc_gather_diff(home_idx, away_idx, strengths):
    info = plsc.get_sparse_core_info()
    num_workers = info.num_cores * info.num_subcores
    bpw = _BATCH // num_workers
    mesh = plsc.VectorSubcoreMesh(core_axis_name="c", subcore_axis_name="s")

    @functools.partial(
        pl.kernel,
        mesh=mesh,
        out_type=jax.ShapeDtypeStruct((_BATCH,), jnp.float32),
        scratch_types=[
            pltpu.VMEM((2 * bpw,), jnp.int32),
            pltpu.VMEM((2 * bpw,), jnp.float32),
            pltpu.SemaphoreType.DMA,
            pltpu.SemaphoreType.DMA,
            pltpu.SemaphoreType.DMA,
        ],
    )
    def k(home_hbm, away_hbm, table_hbm, out_hbm, ix, sv, s1, s2, s3):
        wid = lax.axis_index("s") * info.num_cores + lax.axis_index("c")
        base = wid * bpw
        ci1 = pltpu.async_copy(home_hbm.at[pl.ds(base, bpw)],
                               ix.at[pl.ds(0, bpw)], s2)
        ci2 = pltpu.async_copy(away_hbm.at[pl.ds(base, bpw)],
                               ix.at[pl.ds(bpw, bpw)], s3)
        ci1.wait()
        ci2.wait()
        pltpu.async_copy(table_hbm.at[ix], sv, s1).wait()
        for i in range(bpw // 16):
            sl = pl.ds(i * 16, 16)
            sl2 = pl.ds(bpw + i * 16, 16)
            sv[sl] = sv[sl] - sv[sl2]
        pltpu.sync_copy(sv.at[pl.ds(0, bpw)], out_hbm.at[pl.ds(base, bpw)])

    return k(home_idx, away_idx, strengths)


def _matvec_body(xt_ref, b_ref, m_ref, o_ref):
    p = xt_ref[...] * b_ref[...][:, None]
    s = jnp.sum(p, axis=0)
    o_ref[...] = s + m_ref[0]


def _tc_matvec(XT, beta, mu):
    bm = 8192
    return pl.pallas_call(
        _matvec_body,
        grid=(_BATCH // bm,),
        in_specs=[
            pl.BlockSpec((_FEATS, bm), lambda i: (0, i)),
            pl.BlockSpec((_FEATS,), lambda i: (0,)),
            pl.BlockSpec(memory_space=pltpu.SMEM),
        ],
        out_specs=pl.BlockSpec((bm,), lambda i: (i,)),
        out_shape=jax.ShapeDtypeStruct((_BATCH,), jnp.float32),
    )(XT, beta, mu)


def _combine_body(y_ref, d_ref, o_ref):
    o_ref[...] = y_ref[...] + d_ref[...]


def _tc_combine(y, d):
    return pl.pallas_call(
        _combine_body,
        out_shape=jax.ShapeDtypeStruct((_BATCH,), jnp.float32),
    )(y, d)


@jax.jit
def kernel(home_idx, away_idx, X, strengths, beta, mu):
    d = _sc_gather_diff(home_idx, away_idx, strengths)
    y = _tc_matvec(X.T, beta, mu)
    return _tc_combine(y, d)


# speedup vs baseline: 1.0196x; 1.0196x over previous
import functools

import jax
import jax.numpy as jnp
from jax import lax
from jax.experimental import pallas as pl
from jax.experimental.pallas import tpu as pltpu
from jax.experimental.pallas import tpu_sc as plsc

_BATCH = 16384
_FEATS = 64


def _sc_gather_diff(home_idx, away_idx, strengths):
    info = plsc.get_sparse_core_info()
    num_workers = info.num_cores * info.num_subcores
    bpw = _BATCH // num_workers
    mesh = plsc.VectorSubcoreMesh(core_axis_name="c", subcore_axis_name="s")

    @functools.partial(
        pl.kernel,
        mesh=mesh,
        out_type=jax.ShapeDtypeStruct((_BATCH,), jnp.float32),
        scratch_types=[
            pltpu.VMEM((2 * bpw,), jnp.int32),
            pltpu.VMEM((2 * bpw,), jnp.float32),
            pltpu.SemaphoreType.DMA,
            pltpu.SemaphoreType.DMA,
            pltpu.SemaphoreType.DMA,
            pltpu.SemaphoreType.DMA,
            pltpu.SemaphoreType.DMA,
            pltpu.SemaphoreType.DMA,
            pltpu.SemaphoreType.DMA,
            pltpu.SemaphoreType.DMA,
            pltpu.SemaphoreType.DMA,
            pltpu.SemaphoreType.DMA,
        ],
    )
    def k(home_hbm, away_hbm, table_hbm, out_hbm, ix, sv,
          u0, u1, u2, u3, g0, g1, g2, g3, w0, w1):
        wid = lax.axis_index("s") * info.num_cores + lax.axis_index("c")
        base = wid * bpw
        half = bpw // 2
        cu0 = pltpu.async_copy(home_hbm.at[pl.ds(base, half)],
                               ix.at[pl.ds(0, half)], u0)
        cu1 = pltpu.async_copy(away_hbm.at[pl.ds(base, half)],
                               ix.at[pl.ds(bpw, half)], u1)
        cu2 = pltpu.async_copy(home_hbm.at[pl.ds(base + half, half)],
                               ix.at[pl.ds(half, half)], u2)
        cu3 = pltpu.async_copy(away_hbm.at[pl.ds(base + half, half)],
                               ix.at[pl.ds(bpw + half, half)], u3)
        cu0.wait()
        cg0 = pltpu.async_copy(table_hbm.at[ix.at[pl.ds(0, half)]],
                               sv.at[pl.ds(0, half)], g0)
        cu1.wait()
        cg1 = pltpu.async_copy(table_hbm.at[ix.at[pl.ds(bpw, half)]],
                               sv.at[pl.ds(bpw, half)], g1)
        cu2.wait()
        cg2 = pltpu.async_copy(table_hbm.at[ix.at[pl.ds(half, half)]],
                               sv.at[pl.ds(half, half)], g2)
        cu3.wait()
        cg3 = pltpu.async_copy(table_hbm.at[ix.at[pl.ds(bpw + half, half)]],
                               sv.at[pl.ds(bpw + half, half)], g3)
        cg0.wait()
        cg1.wait()
        for i in range(half // 16):
            sl = pl.ds(i * 16, 16)
            sl2 = pl.ds(bpw + i * 16, 16)
            sv[sl] = sv[sl] - sv[sl2]
        cw0 = pltpu.async_copy(sv.at[pl.ds(0, half)],
                               out_hbm.at[pl.ds(base, half)], w0)
        cg2.wait()
        cg3.wait()
        for i in range(half // 16):
            sl = pl.ds(half + i * 16, 16)
            sl2 = pl.ds(bpw + half + i * 16, 16)
            sv[sl] = sv[sl] - sv[sl2]
        cw1 = pltpu.async_copy(sv.at[pl.ds(half, half)],
                               out_hbm.at[pl.ds(base + half, half)], w1)
        cw0.wait()
        cw1.wait()

    return k(home_idx, away_idx, strengths)


def _matvec_body(xt_ref, b_ref, m_ref, o_ref):
    p = xt_ref[...] * b_ref[...][:, None]
    s = jnp.sum(p, axis=0)
    o_ref[...] = s + m_ref[0]


def _tc_matvec(XT, beta, mu):
    bm = 8192
    return pl.pallas_call(
        _matvec_body,
        grid=(_BATCH // bm,),
        in_specs=[
            pl.BlockSpec((_FEATS, bm), lambda i: (0, i)),
            pl.BlockSpec((_FEATS,), lambda i: (0,)),
            pl.BlockSpec(memory_space=pltpu.SMEM),
        ],
        out_specs=pl.BlockSpec((bm,), lambda i: (i,)),
        out_shape=jax.ShapeDtypeStruct((_BATCH,), jnp.float32),
    )(XT, beta, mu)


def _combine_body(y_ref, d_ref, o_ref):
    o_ref[...] = y_ref[...] + d_ref[...]


def _tc_combine(y, d):
    return pl.pallas_call(
        _combine_body,
        out_shape=jax.ShapeDtypeStruct((_BATCH,), jnp.float32),
    )(y, d)


@jax.jit
def kernel(home_idx, away_idx, X, strengths, beta, mu):
    d = _sc_gather_diff(home_idx, away_idx, strengths)
    y = _tc_matvec(X.T, beta, mu)
    return _tc_combine(y, d)
